# Initial kernel scaffold; baseline (speedup 1.0000x reference)
import jax
import jax.numpy as jnp
from jax.experimental import pallas as pl


def kernel(means, precisions, targets):
    raise NotImplementedError("write your pallas kernel here")


# speedup vs baseline: 21.6476x; 21.6476x over previous
import math

import jax
import jax.numpy as jnp
from jax import lax
from jax.experimental import pallas as pl

NUM_CLASSES = 128
LOG_2PI = math.log(2.0 * math.pi)


def _split_dot(oh, x):
    x_hi = x.astype(jnp.bfloat16)
    x_lo = (x - x_hi.astype(jnp.float32)).astype(jnp.bfloat16)
    dn = (((0,), (0,)), ((), ()))
    hi = lax.dot_general(oh, x_hi, dn, preferred_element_type=jnp.float32)
    lo = lax.dot_general(oh, x_lo, dn, preferred_element_type=jnp.float32)
    return hi + lo


def _body(t_ref, p_ref, m_ref, pm_out, pp_out, ln_out):
    p = p_ref[0]
    m = m_ref[0]
    t = t_ref[0]
    n_ex, d = p.shape
    cls = lax.broadcasted_iota(jnp.int32, (n_ex, NUM_CLASSES), 1)
    mask = t.reshape(n_ex, 1) == cls
    oh = mask.astype(jnp.bfloat16)

    pm = p * m
    seg_p = _split_dot(oh, p)
    seg_pm = _split_dot(oh, pm)

    r_pmm = jnp.sum(pm * m, axis=1, keepdims=True)
    r_lp = jnp.sum(jnp.log(p), axis=1, keepdims=True)
    seg_pmm = jnp.sum(jnp.where(mask, r_pmm, 0.0), axis=0, keepdims=True)
    seg_rlp = jnp.sum(jnp.where(mask, r_lp, 0.0), axis=0, keepdims=True)
    counts = jnp.sum(mask.astype(jnp.float32), axis=0, keepdims=True)
    ns = jnp.maximum(counts, 1.0)

    mean = seg_pm * jnp.reciprocal(seg_p)
    pp_out[0] = seg_p
    pm_out[0] = mean

    expo = 0.5 * (jnp.sum(seg_pm * mean, axis=1).reshape(1, NUM_CLASSES)
                  - seg_pmm)
    log_det = 0.5 * (seg_rlp
                     - jnp.sum(jnp.log(seg_p), axis=1).reshape(1, NUM_CLASSES))
    ln_out[0] = 0.5 * (1.0 - ns) * (d * LOG_2PI) + log_det + expo


def kernel(means, precisions, targets):
    b, n, d = means.shape
    t3 = targets.reshape(b, 1, n)
    pm_o, pp_o, ln_o = pl.pallas_call(
        _body,
        grid=(b,),
        in_specs=[
            pl.BlockSpec((1, 1, n), lambda i: (i, 0, 0)),
            pl.BlockSpec((1, n, d), lambda i: (i, 0, 0)),
            pl.BlockSpec((1, n, d), lambda i: (i, 0, 0)),
        ],
        out_specs=[
            pl.BlockSpec((1, NUM_CLASSES, d), lambda i: (i, 0, 0)),
            pl.BlockSpec((1, NUM_CLASSES, d), lambda i: (i, 0, 0)),
            pl.BlockSpec((1, 1, NUM_CLASSES), lambda i: (i, 0, 0)),
        ],
        out_shape=[
            jax.ShapeDtypeStruct((b, NUM_CLASSES, d), jnp.float32),
            jax.ShapeDtypeStruct((b, NUM_CLASSES, d), jnp.float32),
            jax.ShapeDtypeStruct((b, 1, NUM_CLASSES), jnp.float32),
        ],
    )(t3, precisions, means)
    return (pm_o, pp_o, ln_o.reshape(b, NUM_CLASSES))
